# baseline (speedup 1.0000x reference)
import functools

import jax
import jax.numpy as jnp
from jax import lax
from jax.experimental import pallas as pl
from jax.experimental.pallas import tpu as pltpu
from jax.experimental.pallas import tpu_sc as plsc


@functools.lru_cache(maxsize=None)
def _build_sc_copy(n_rows, n_cols, row_offset, dtype_name):
    dtype = jnp.dtype(dtype_name)
    info = plsc.get_sparse_core_info()
    num_workers = info.num_cores * info.num_subcores
    assert n_rows % num_workers == 0, (n_rows, num_workers)
    rows_per = n_rows // num_workers

    mesh = plsc.VectorSubcoreMesh(core_axis_name="c", subcore_axis_name="s")

    @functools.partial(
        pl.kernel,
        mesh=mesh,
        out_type=jax.ShapeDtypeStruct((n_rows, n_cols), dtype),
    )
    def sc_copy(w_hbm, out_hbm):
        wid = lax.axis_index("s") * info.num_cores + lax.axis_index("c")
        base = wid * rows_per
        pltpu.sync_copy(
            w_hbm.at[pl.ds(row_offset + base, rows_per)],
            out_hbm.at[pl.ds(base, rows_per)],
        )

    return sc_copy


def kernel(input, weights):
    bsz, seq_len = input.shape
    origin_shift = weights.shape[0] // 2
    start = int(-seq_len / 2)
    end = round(seq_len / 2 + 1e-05)
    n_out = end - start
    row_offset = start + origin_shift
    sc_copy = _build_sc_copy(
        n_out, weights.shape[1], row_offset, weights.dtype.name
    )
    return sc_copy(weights)


# speedup vs baseline: 24.0790x; 24.0790x over previous
import functools

import jax
import jax.numpy as jnp
from jax import lax
from jax.experimental import pallas as pl
from jax.experimental.pallas import tpu as pltpu
from jax.experimental.pallas import tpu_sc as plsc


@functools.lru_cache(maxsize=None)
def _build_sc_copy(n_rows, n_cols, row_offset, dtype_name):
    dtype = jnp.dtype(dtype_name)
    info = plsc.get_sparse_core_info()
    num_workers = info.num_cores * info.num_subcores
    assert n_rows % num_workers == 0, (n_rows, num_workers)
    rows_per = n_rows // num_workers

    mesh = plsc.VectorSubcoreMesh(core_axis_name="c", subcore_axis_name="s")

    nbuf = 4
    chunk = rows_per
    while chunk > 16 or (nbuf * chunk * n_cols * dtype.itemsize) > 400_000:
        assert chunk % 2 == 0, (rows_per, chunk)
        chunk //= 2
    nch = rows_per // chunk

    @functools.partial(
        pl.kernel,
        mesh=mesh,
        out_type=jax.ShapeDtypeStruct((n_rows, n_cols), dtype),
        scratch_types=[
            pltpu.VMEM((nbuf, chunk, n_cols), dtype),
            pltpu.SemaphoreType.DMA((nbuf,)),
            pltpu.SemaphoreType.DMA((nbuf,)),
        ],
    )
    def sc_copy(w_hbm, out_hbm, buf, in_sems, out_sems):
        wid = lax.axis_index("s") * info.num_cores + lax.axis_index("c")
        base = wid * rows_per

        def in_copy(i, b):
            return pltpu.make_async_copy(
                w_hbm.at[pl.ds(row_offset + base + i * chunk, chunk)],
                buf.at[b],
                in_sems.at[b],
            )

        def out_copy(i, b):
            return pltpu.make_async_copy(
                buf.at[b],
                out_hbm.at[pl.ds(base + i * chunk, chunk)],
                out_sems.at[b],
            )

        for b in range(min(nbuf, nch)):
            in_copy(b, b).start()
        for i in range(nch):
            b = i % nbuf
            in_copy(i, b).wait()
            out_copy(i, b).start()
            j = i + nbuf
            if j < nch:
                out_copy(i, b).wait()
                in_copy(j, b).start()
        for i in range(max(nch - nbuf, 0), nch):
            out_copy(i, i % nbuf).wait()

    return sc_copy


def kernel(input, weights):
    bsz, seq_len = input.shape
    origin_shift = weights.shape[0] // 2
    start = int(-seq_len / 2)
    end = round(seq_len / 2 + 1e-05)
    n_out = end - start
    row_offset = start + origin_shift
    sc_copy = _build_sc_copy(
        n_out, weights.shape[1], row_offset, weights.dtype.name
    )
    return sc_copy(weights)


# speedup vs baseline: 24.8554x; 1.0322x over previous
import functools

import jax
import jax.numpy as jnp
from jax import lax
from jax.experimental import pallas as pl
from jax.experimental.pallas import tpu as pltpu
from jax.experimental.pallas import tpu_sc as plsc


@functools.lru_cache(maxsize=None)
def _build_sc_copy(n_rows, n_cols, row_offset, dtype_name):
    dtype = jnp.dtype(dtype_name)
    info = plsc.get_sparse_core_info()
    num_workers = info.num_cores * info.num_subcores
    assert n_rows % num_workers == 0, (n_rows, num_workers)
    rows_per = n_rows // num_workers

    mesh = plsc.VectorSubcoreMesh(core_axis_name="c", subcore_axis_name="s")

    nbuf = 3
    chunk = rows_per
    while chunk > 32 or (nbuf * chunk * n_cols * dtype.itemsize) > 400_000:
        assert chunk % 2 == 0, (rows_per, chunk)
        chunk //= 2
    nch = rows_per // chunk

    @functools.partial(
        pl.kernel,
        mesh=mesh,
        out_type=jax.ShapeDtypeStruct((n_rows, n_cols), dtype),
        scratch_types=[
            pltpu.VMEM((nbuf, chunk, n_cols), dtype),
            pltpu.SemaphoreType.DMA((nbuf,)),
            pltpu.SemaphoreType.DMA((nbuf,)),
        ],
    )
    def sc_copy(w_hbm, out_hbm, buf, in_sems, out_sems):
        wid = lax.axis_index("s") * info.num_cores + lax.axis_index("c")
        base = wid * rows_per

        def in_copy(i, b):
            return pltpu.make_async_copy(
                w_hbm.at[pl.ds(row_offset + base + i * chunk, chunk)],
                buf.at[b],
                in_sems.at[b],
            )

        def out_copy(i, b):
            return pltpu.make_async_copy(
                buf.at[b],
                out_hbm.at[pl.ds(base + i * chunk, chunk)],
                out_sems.at[b],
            )

        for b in range(min(nbuf, nch)):
            in_copy(b, b).start()
        for i in range(nch):
            b = i % nbuf
            in_copy(i, b).wait()
            out_copy(i, b).start()
            j = i + nbuf
            if j < nch:
                out_copy(i, b).wait()
                in_copy(j, b).start()
        for i in range(max(nch - nbuf, 0), nch):
            out_copy(i, i % nbuf).wait()

    return sc_copy


def kernel(input, weights):
    bsz, seq_len = input.shape
    origin_shift = weights.shape[0] // 2
    start = int(-seq_len / 2)
    end = round(seq_len / 2 + 1e-05)
    n_out = end - start
    row_offset = start + origin_shift
    sc_copy = _build_sc_copy(
        n_out, weights.shape[1], row_offset, weights.dtype.name
    )
    return sc_copy(weights)
